# Initial kernel scaffold; baseline (speedup 1.0000x reference)
import jax
import jax.numpy as jnp
from jax.experimental import pallas as pl


def kernel(x, edge_index, batch, emb, W1, b1, W2, b2):
    raise NotImplementedError("write your pallas kernel here")


# speedup vs baseline: 81.4098x; 81.4098x over previous
import functools

import jax
import jax.numpy as jnp
import numpy as np
from jax import lax
from jax.experimental import pallas as pl
from jax.experimental.pallas import tpu as pltpu
from jax.experimental.pallas import tpu_sc as plsc

N = 10000
E = 320000
NFEAT = 128
NHID = 64
NCLASS = 10
B = 256
EMB_ROWS = 5000
EMB_DIM = 32
NLAYER = 2

NT = 16
NPAD = 10240
NODES_PT = NPAD // NT
EPT = E // NT
BINS = 512
GPT = B // NT
VL = 16

_M1 = np.int32(np.uint32(0x85EBCA6B))
_M2 = np.int32(np.uint32(0xC2B2AE35))
_GOLD = np.int32(np.uint32(0x9E3779B1))


def _mix32(h):
    h = h * _M1
    h = h ^ lax.shift_right_logical(h, 13)
    h = h * _M2
    h = h ^ lax.shift_right_logical(h, 16)
    return h


def _argmax_body(x_ref, o_ref):
    o_ref[...] = jnp.argmax(x_ref[...], axis=-1).astype(jnp.int32)


def _colors_tc(x):
    return pl.pallas_call(
        _argmax_body,
        out_shape=jax.ShapeDtypeStruct((N,), jnp.int32),
    )(x)


def _sc_body(colors_hbm, src_hbm, dst_hbm, batch_hbm, emb_hbm, out_hbm,
             colors_v, mixed_v, agg_v, agg2_v, tmp_v, src_v, dst_v,
             batch_v, bins_v, allbins_v, gidx_v, rows_v,
             spm_agg, spm_colors, spm_bins, sem):
    wid = lax.axis_index("s")
    nbase = pl.multiple_of(wid * NODES_PT, 8)
    ebase = pl.multiple_of(wid * EPT, 8)

    pltpu.sync_copy(colors_hbm, colors_v)
    pltpu.sync_copy(src_hbm.at[pl.ds(ebase, EPT)], src_v)
    pltpu.sync_copy(dst_hbm.at[pl.ds(ebase, EPT)], dst_v)
    pltpu.sync_copy(batch_hbm.at[pl.ds(nbase, NODES_PT)], batch_v)

    for _layer in range(NLAYER):
        def mix_body(i, _):
            o = pl.multiple_of(i * VL, VL)
            mixed_v[pl.ds(o, VL)] = _mix32(colors_v[pl.ds(o, VL)])
            return 0
        lax.fori_loop(0, NPAD // VL, mix_body, 0)

        def zero_body(i, _):
            o = pl.multiple_of(i * VL, VL)
            agg_v[pl.ds(o, VL)] = jnp.zeros((VL,), jnp.int32)
            return 0
        lax.fori_loop(0, NPAD // VL, zero_body, 0)

        def edge_body(i, _):
            o = pl.multiple_of(i * VL, VL)
            s = src_v[pl.ds(o, VL)]
            d = dst_v[pl.ds(o, VL)]
            vals = plsc.load_gather(mixed_v, [s])
            plsc.addupdate_scatter(agg_v, [d], vals)
            return 0
        lax.fori_loop(0, EPT // VL, edge_body, 0)

        pltpu.sync_copy(agg_v, spm_agg.at[wid])
        plsc.subcore_barrier()

        for t in range(NT):
            pltpu.sync_copy(spm_agg.at[t, pl.ds(nbase, NODES_PT)], tmp_v)

            def red_body(i, _, first=(t == 0)):
                o = pl.multiple_of(i * VL, VL)
                v = tmp_v[pl.ds(o, VL)]
                if first:
                    agg2_v[pl.ds(o, VL)] = v
                else:
                    agg2_v[pl.ds(o, VL)] = agg2_v[pl.ds(o, VL)] + v
                return 0
            lax.fori_loop(0, NODES_PT // VL, red_body, 0)

        def upd_body(i, _):
            o = pl.multiple_of(i * VL, VL)
            c = colors_v[pl.ds(nbase + o, VL)]
            a = agg2_v[pl.ds(o, VL)]
            colors_v[pl.ds(nbase + o, VL)] = _mix32((c * _GOLD) ^ a)
            return 0
        lax.fori_loop(0, NODES_PT // VL, upd_body, 0)

        pltpu.sync_copy(colors_v.at[pl.ds(nbase, NODES_PT)],
                        spm_colors.at[pl.ds(nbase, NODES_PT)])
        plsc.subcore_barrier()
        pltpu.sync_copy(spm_colors, colors_v)

    def binzero_body(i, _):
        o = pl.multiple_of(i * VL, VL)
        bins_v[pl.ds(o, VL)] = jnp.zeros((VL,), jnp.int32)
        return 0
    lax.fori_loop(0, BINS // VL, binzero_body, 0)

    def bin_body(i, _):
        o = pl.multiple_of(i * VL, VL)
        c = colors_v[pl.ds(nbase + o, VL)]
        b = batch_v[pl.ds(o, VL)]
        plsc.addupdate_scatter(bins_v, [b], _mix32(c))
        return 0
    lax.fori_loop(0, NODES_PT // VL, bin_body, 0)

    pltpu.sync_copy(bins_v, spm_bins.at[wid])
    plsc.subcore_barrier()
    pltpu.sync_copy(spm_bins, allbins_v)

    gbase = pl.multiple_of(wid * GPT, 8)
    ghash = jnp.zeros((GPT,), jnp.int32)
    for t in range(NT):
        ghash = ghash + allbins_v[t, pl.ds(gbase, GPT)]

    gu = plsc.bitcast(ghash, jnp.uint32)
    gidx = (gu % jnp.uint32(EMB_ROWS)).astype(jnp.int32)
    gidx_v[...] = gidx

    pltpu.async_copy(emb_hbm.at[gidx_v], rows_v, sem).wait()
    pltpu.sync_copy(rows_v, out_hbm.at[pl.ds(gbase, GPT)])


def _sc_call(colors_pad, src, dst, batch_pad, emb):
    mesh = plsc.VectorSubcoreMesh(core_axis_name="c", subcore_axis_name="s",
                                  num_cores=1)
    fn = pl.kernel(
        _sc_body,
        out_type=jax.ShapeDtypeStruct((B, EMB_DIM), jnp.float32),
        mesh=mesh,
        scratch_types=[
            pltpu.VMEM((NPAD,), jnp.int32),
            pltpu.VMEM((NPAD,), jnp.int32),
            pltpu.VMEM((NPAD,), jnp.int32),
            pltpu.VMEM((NODES_PT,), jnp.int32),
            pltpu.VMEM((NODES_PT,), jnp.int32),
            pltpu.VMEM((EPT,), jnp.int32),
            pltpu.VMEM((EPT,), jnp.int32),
            pltpu.VMEM((NODES_PT,), jnp.int32),
            pltpu.VMEM((BINS,), jnp.int32),
            pltpu.VMEM((NT, BINS), jnp.int32),
            pltpu.VMEM((GPT,), jnp.int32),
            pltpu.VMEM((GPT, EMB_DIM), jnp.float32),
            pltpu.VMEM_SHARED((NT, NPAD), jnp.int32),
            pltpu.VMEM_SHARED((NPAD,), jnp.int32),
            pltpu.VMEM_SHARED((NT, BINS), jnp.int32),
            pltpu.SemaphoreType.DMA,
        ],
        compiler_params=pltpu.CompilerParams(needs_layout_passes=False,
                                             use_tc_tiling_on_sc=False),
    )
    return fn(colors_pad, src, dst, batch_pad, emb)


def _mlp_body(gx_ref, w1_ref, b1_ref, w2_ref, b2_ref, o_ref):
    h = jnp.dot(gx_ref[...], w1_ref[...],
                preferred_element_type=jnp.float32) + b1_ref[...]
    h = jnp.where(h > 0, h, jnp.float32(0.01) * h)
    logits = jnp.dot(h, w2_ref[...],
                     preferred_element_type=jnp.float32) + b2_ref[...]
    m = jnp.max(logits, axis=1, keepdims=True)
    s = logits - m
    lse = jnp.log(jnp.sum(jnp.exp(s), axis=1, keepdims=True))
    o_ref[...] = s - lse


def _mlp_tc(gx, W1, b1, W2, b2):
    return pl.pallas_call(
        _mlp_body,
        out_shape=jax.ShapeDtypeStruct((B, NCLASS), jnp.float32),
    )(gx, W1, b1, W2, b2)


@jax.jit
def kernel(x, edge_index, batch, emb, W1, b1, W2, b2):
    colors = _colors_tc(x)
    colors_pad = jnp.concatenate(
        [colors, jnp.zeros((NPAD - N,), jnp.int32)])
    batch_pad = jnp.concatenate(
        [batch, jnp.full((NPAD - N,), B, jnp.int32)])
    src = edge_index[0]
    dst = edge_index[1]
    gx = _sc_call(colors_pad, src, dst, batch_pad, emb)
    return _mlp_tc(gx, W1, b1, W2, b2)


# speedup vs baseline: 99.7440x; 1.2252x over previous
import functools

import jax
import jax.numpy as jnp
import numpy as np
from jax import lax
from jax.experimental import pallas as pl
from jax.experimental.pallas import tpu as pltpu
from jax.experimental.pallas import tpu_sc as plsc

N = 10000
E = 320000
NFEAT = 128
NHID = 64
NCLASS = 10
B = 256
EMB_ROWS = 5000
EMB_DIM = 32
NLAYER = 2

NT = 16
NPAD = 10240
NODES_PT = NPAD // NT
EPT = E // NT
BINS = 512
GPT = B // NT
VL = 16

_M1 = np.int32(np.uint32(0x85EBCA6B))
_M2 = np.int32(np.uint32(0xC2B2AE35))
_GOLD = np.int32(np.uint32(0x9E3779B1))


def _mix32(h):
    h = h * _M1
    h = h ^ lax.shift_right_logical(h, 13)
    h = h * _M2
    h = h ^ lax.shift_right_logical(h, 16)
    return h


def _unrolled(n_total, unroll, body):
    assert n_total % unroll == 0

    def outer(i, _):
        base = i * (VL * unroll)
        for u in range(unroll):
            body(pl.multiple_of(base + u * VL, VL))
        return 0
    lax.fori_loop(0, n_total // unroll, outer, 0)


def _argmax_body(x_ref, o_ref):
    am = jnp.argmax(x_ref[...], axis=-1).astype(jnp.int32)
    o_ref[...] = jnp.concatenate([am, jnp.zeros((NPAD - N,), jnp.int32)])


def _colors_tc(x):
    return pl.pallas_call(
        _argmax_body,
        out_shape=jax.ShapeDtypeStruct((NPAD,), jnp.int32),
    )(x)


def _sc_body(colors_hbm, src_hbm, dst_hbm, batch_hbm, emb_hbm, out_hbm,
             colors_v, mixed_v, agg_v, red_v, src_v, dst_v,
             batch_v, bins_v, binred_v, gidx_v, rows_v,
             spm_agg, spm_mixed, spm_bins, sem_s, sem_d, sem_b):
    wid = lax.axis_index("s")
    nbase = pl.multiple_of(wid * NODES_PT, 8)
    ebase = pl.multiple_of(wid * EPT, 8)
    gbase = pl.multiple_of(wid * GPT, 8)

    cp_s = pltpu.async_copy(src_hbm.at[pl.ds(ebase, EPT)], src_v, sem_s)
    cp_d = pltpu.async_copy(dst_hbm.at[pl.ds(ebase, EPT)], dst_v, sem_d)
    cp_b = pltpu.async_copy(batch_hbm.at[pl.ds(nbase, NODES_PT)], batch_v,
                            sem_b)
    pltpu.sync_copy(colors_hbm.at[pl.ds(nbase, NODES_PT)], colors_v)

    edges_pending = True
    for _layer in range(NLAYER):
        def mix_body(o):
            mixed_v[pl.ds(nbase + o, VL)] = _mix32(colors_v[pl.ds(o, VL)])
        _unrolled(NODES_PT // VL, 8, mix_body)

        pltpu.sync_copy(mixed_v.at[pl.ds(nbase, NODES_PT)],
                        spm_mixed.at[pl.ds(nbase, NODES_PT)])
        plsc.subcore_barrier()
        pltpu.sync_copy(spm_mixed, mixed_v)

        def zero_body(o):
            agg_v[pl.ds(o, VL)] = jnp.zeros((VL,), jnp.int32)
        _unrolled(NPAD // VL, 8, zero_body)

        if edges_pending:
            cp_s.wait()
            cp_d.wait()
            edges_pending = False

        def edge_body(o):
            s = src_v[pl.ds(o, VL)]
            d = dst_v[pl.ds(o, VL)]
            vals = plsc.load_gather(mixed_v, [s])
            plsc.addupdate_scatter(agg_v, [d], vals)
        _unrolled(EPT // VL, 5, edge_body)

        pltpu.sync_copy(agg_v, spm_agg.at[wid])
        plsc.subcore_barrier()
        for t in range(NT):
            pltpu.sync_copy(spm_agg.at[t, pl.ds(nbase, NODES_PT)],
                            red_v.at[t])

        def upd_body(i, _):
            o = pl.multiple_of(i * VL, VL)
            a = red_v[0, pl.ds(o, VL)]
            for t in range(1, NT):
                a = a + red_v[t, pl.ds(o, VL)]
            c = colors_v[pl.ds(o, VL)]
            colors_v[pl.ds(o, VL)] = _mix32((c * _GOLD) ^ a)
            return 0
        lax.fori_loop(0, NODES_PT // VL, upd_body, 0)

    def binzero_body(o):
        bins_v[pl.ds(o, VL)] = jnp.zeros((VL,), jnp.int32)
    _unrolled(BINS // VL, 8, binzero_body)

    cp_b.wait()

    def bin_body(o):
        c = colors_v[pl.ds(o, VL)]
        b = batch_v[pl.ds(o, VL)]
        plsc.addupdate_scatter(bins_v, [b], _mix32(c))
    _unrolled(NODES_PT // VL, 8, bin_body)

    pltpu.sync_copy(bins_v, spm_bins.at[wid])
    plsc.subcore_barrier()
    for t in range(NT):
        pltpu.sync_copy(spm_bins.at[t, pl.ds(gbase, GPT)], binred_v.at[t])

    ghash = binred_v[0, pl.ds(0, GPT)]
    for t in range(1, NT):
        ghash = ghash + binred_v[t, pl.ds(0, GPT)]

    gu = plsc.bitcast(ghash, jnp.uint32)
    gidx_v[...] = (gu % jnp.uint32(EMB_ROWS)).astype(jnp.int32)

    pltpu.async_copy(emb_hbm.at[gidx_v], rows_v, sem_s).wait()
    pltpu.sync_copy(rows_v, out_hbm.at[pl.ds(gbase, GPT)])


def _sc_call(colors_pad, src, dst, batch_pad, emb):
    mesh = plsc.VectorSubcoreMesh(core_axis_name="c", subcore_axis_name="s",
                                  num_cores=1)
    fn = pl.kernel(
        _sc_body,
        out_type=jax.ShapeDtypeStruct((B, EMB_DIM), jnp.float32),
        mesh=mesh,
        scratch_types=[
            pltpu.VMEM((NODES_PT,), jnp.int32),
            pltpu.VMEM((NPAD,), jnp.int32),
            pltpu.VMEM((NPAD,), jnp.int32),
            pltpu.VMEM((NT, NODES_PT), jnp.int32),
            pltpu.VMEM((EPT,), jnp.int32),
            pltpu.VMEM((EPT,), jnp.int32),
            pltpu.VMEM((NODES_PT,), jnp.int32),
            pltpu.VMEM((BINS,), jnp.int32),
            pltpu.VMEM((NT, GPT), jnp.int32),
            pltpu.VMEM((GPT,), jnp.int32),
            pltpu.VMEM((GPT, EMB_DIM), jnp.float32),
            pltpu.VMEM_SHARED((NT, NPAD), jnp.int32),
            pltpu.VMEM_SHARED((NPAD,), jnp.int32),
            pltpu.VMEM_SHARED((NT, BINS), jnp.int32),
            pltpu.SemaphoreType.DMA,
            pltpu.SemaphoreType.DMA,
            pltpu.SemaphoreType.DMA,
        ],
        compiler_params=pltpu.CompilerParams(needs_layout_passes=False,
                                             use_tc_tiling_on_sc=False),
    )
    return fn(colors_pad, src, dst, batch_pad, emb)


def _mlp_body(gx_ref, w1_ref, b1_ref, w2_ref, b2_ref, o_ref):
    h = jnp.dot(gx_ref[...], w1_ref[...],
                preferred_element_type=jnp.float32) + b1_ref[...]
    h = jnp.where(h > 0, h, jnp.float32(0.01) * h)
    logits = jnp.dot(h, w2_ref[...],
                     preferred_element_type=jnp.float32) + b2_ref[...]
    m = jnp.max(logits, axis=1, keepdims=True)
    s = logits - m
    lse = jnp.log(jnp.sum(jnp.exp(s), axis=1, keepdims=True))
    o_ref[...] = s - lse


def _mlp_tc(gx, W1, b1, W2, b2):
    return pl.pallas_call(
        _mlp_body,
        out_shape=jax.ShapeDtypeStruct((B, NCLASS), jnp.float32),
    )(gx, W1, b1, W2, b2)


@jax.jit
def kernel(x, edge_index, batch, emb, W1, b1, W2, b2):
    colors_pad = _colors_tc(x)
    batch_pad = jnp.concatenate(
        [batch, jnp.full((NPAD - N,), B, jnp.int32)])
    gx = _sc_call(colors_pad, edge_index[0], edge_index[1], batch_pad, emb)
    return _mlp_tc(gx, W1, b1, W2, b2)


# speedup vs baseline: 117.5624x; 1.1786x over previous
import functools

import jax
import jax.numpy as jnp
import numpy as np
from jax import lax
from jax.experimental import pallas as pl
from jax.experimental.pallas import tpu as pltpu
from jax.experimental.pallas import tpu_sc as plsc

N = 10000
E = 320000
NFEAT = 128
NHID = 64
NCLASS = 10
B = 256
EMB_ROWS = 5000
EMB_DIM = 32
NLAYER = 2

NT = 16
NPAD = 10240
NODES_PT = NPAD // NT
EPT = E // NT
BINS = 512
GPT = B // NT
VL = 16

_M1 = np.int32(np.uint32(0x85EBCA6B))
_M2 = np.int32(np.uint32(0xC2B2AE35))
_GOLD = np.int32(np.uint32(0x9E3779B1))


def _mix32(h):
    h = h * _M1
    h = h ^ lax.shift_right_logical(h, 13)
    h = h * _M2
    h = h ^ lax.shift_right_logical(h, 16)
    return h


def _unrolled(n_total, unroll, body):
    assert n_total % unroll == 0

    def outer(i, _):
        base = i * (VL * unroll)
        for u in range(unroll):
            body(pl.multiple_of(base + u * VL, VL))
        return 0
    lax.fori_loop(0, n_total // unroll, outer, 0)


def _argmax_body(x_ref, o_ref):
    am = jnp.argmax(x_ref[...], axis=-1).astype(jnp.int32)
    o_ref[...] = jnp.concatenate([am, jnp.zeros((NPAD - N,), jnp.int32)])


def _colors_tc(x):
    return pl.pallas_call(
        _argmax_body,
        out_shape=jax.ShapeDtypeStruct((NPAD,), jnp.int32),
    )(x)


def _sc_body(colors_hbm, edge_hbm, batch_hbm, emb_hbm, out_hbm,
             colors_v, mixed_v, agg_v, red_v, src_v, dst_v,
             batch_v, bins_v, binred_v, gidx_v, rows_v,
             spm_agg, spm_mixed, spm_bins, sem_s, sem_d, sem_b):
    wid = lax.axis_index("s")
    nbase = pl.multiple_of(wid * NODES_PT, 8)
    sbase = pl.multiple_of(wid * EPT, 8)
    dbase = pl.multiple_of(E + wid * EPT, 8)
    gbase = pl.multiple_of(wid * GPT, 8)

    cp_s = pltpu.async_copy(edge_hbm.at[pl.ds(sbase, EPT)], src_v, sem_s)
    cp_d = pltpu.async_copy(edge_hbm.at[pl.ds(dbase, EPT)], dst_v, sem_d)

    last = NT - 1

    @pl.when(wid < last)
    def _():
        pltpu.sync_copy(batch_hbm.at[pl.ds(nbase, NODES_PT)], batch_v)

    @pl.when(wid == last)
    def _():
        tail = N - last * NODES_PT
        pltpu.sync_copy(batch_hbm.at[pl.ds(last * NODES_PT, tail)],
                        batch_v.at[pl.ds(0, tail)])
        for u in range(tail // VL, NODES_PT // VL):
            batch_v[pl.ds(u * VL, VL)] = jnp.full((VL,), B, jnp.int32)

    pltpu.sync_copy(colors_hbm.at[pl.ds(nbase, NODES_PT)], colors_v)

    edges_pending = True
    for _layer in range(NLAYER):
        with jax.named_scope("mixshare"):
            def mix_body(o):
                mixed_v[pl.ds(nbase + o, VL)] = _mix32(colors_v[pl.ds(o, VL)])
            _unrolled(NODES_PT // VL, 8, mix_body)

            pltpu.sync_copy(mixed_v.at[pl.ds(nbase, NODES_PT)],
                            spm_mixed.at[pl.ds(nbase, NODES_PT)])
            plsc.subcore_barrier()
            pltpu.sync_copy(spm_mixed, mixed_v)

        with jax.named_scope("zero"):
            def zero_body(o):
                agg_v[pl.ds(o, VL)] = jnp.zeros((VL,), jnp.int32)
            _unrolled(NPAD // VL, 8, zero_body)

        if edges_pending:
            cp_s.wait()
            cp_d.wait()
            edges_pending = False

        with jax.named_scope("edges"):
            def edge_body(o):
                s = src_v[pl.ds(o, VL)]
                d = dst_v[pl.ds(o, VL)]
                vals = plsc.load_gather(mixed_v, [s])
                plsc.addupdate_scatter(agg_v, [d], vals)
            _unrolled(EPT // VL, 5, edge_body)

        with jax.named_scope("reduce"):
            pltpu.sync_copy(agg_v, spm_agg.at[wid])
            plsc.subcore_barrier()
            cps = [pltpu.async_copy(spm_agg.at[t, pl.ds(nbase, NODES_PT)],
                                    red_v.at[t], sem_b)
                   for t in range(NT)]
            for cp in cps:
                cp.wait()

            def upd_body(i, _):
                o = pl.multiple_of(i * VL, VL)
                a = red_v[0, pl.ds(o, VL)]
                for t in range(1, NT):
                    a = a + red_v[t, pl.ds(o, VL)]
                c = colors_v[pl.ds(o, VL)]
                colors_v[pl.ds(o, VL)] = _mix32((c * _GOLD) ^ a)
                return 0
            lax.fori_loop(0, NODES_PT // VL, upd_body, 0)

    with jax.named_scope("bins"):
        def binzero_body(o):
            bins_v[pl.ds(o, VL)] = jnp.zeros((VL,), jnp.int32)
        _unrolled(BINS // VL, 8, binzero_body)

        def bin_body(o):
            c = colors_v[pl.ds(o, VL)]
            b = batch_v[pl.ds(o, VL)]
            plsc.addupdate_scatter(bins_v, [b], _mix32(c))
        _unrolled(NODES_PT // VL, 8, bin_body)

        pltpu.sync_copy(bins_v, spm_bins.at[wid])
        plsc.subcore_barrier()
        cps = [pltpu.async_copy(spm_bins.at[t, pl.ds(gbase, GPT)],
                                binred_v.at[t], sem_b)
               for t in range(NT)]
        for cp in cps:
            cp.wait()

        ghash = binred_v[0, pl.ds(0, GPT)]
        for t in range(1, NT):
            ghash = ghash + binred_v[t, pl.ds(0, GPT)]

        gu = plsc.bitcast(ghash, jnp.uint32)
        gidx_v[...] = (gu % jnp.uint32(EMB_ROWS)).astype(jnp.int32)

    with jax.named_scope("emb"):
        pltpu.async_copy(emb_hbm.at[gidx_v], rows_v, sem_s).wait()
        pltpu.sync_copy(rows_v, out_hbm.at[pl.ds(gbase, GPT)])


def _sc_call(colors_pad, edge_flat, batch, emb):
    mesh = plsc.VectorSubcoreMesh(core_axis_name="c", subcore_axis_name="s",
                                  num_cores=1)
    fn = pl.kernel(
        _sc_body,
        out_type=jax.ShapeDtypeStruct((B, EMB_DIM), jnp.float32),
        mesh=mesh,
        scratch_types=[
            pltpu.VMEM((NODES_PT,), jnp.int32),
            pltpu.VMEM((NPAD,), jnp.int32),
            pltpu.VMEM((NPAD,), jnp.int32),
            pltpu.VMEM((NT, NODES_PT), jnp.int32),
            pltpu.VMEM((EPT,), jnp.int32),
            pltpu.VMEM((EPT,), jnp.int32),
            pltpu.VMEM((NODES_PT,), jnp.int32),
            pltpu.VMEM((BINS,), jnp.int32),
            pltpu.VMEM((NT, GPT), jnp.int32),
            pltpu.VMEM((GPT,), jnp.int32),
            pltpu.VMEM((GPT, EMB_DIM), jnp.float32),
            pltpu.VMEM_SHARED((NT, NPAD), jnp.int32),
            pltpu.VMEM_SHARED((NPAD,), jnp.int32),
            pltpu.VMEM_SHARED((NT, BINS), jnp.int32),
            pltpu.SemaphoreType.DMA,
            pltpu.SemaphoreType.DMA,
            pltpu.SemaphoreType.DMA,
        ],
        compiler_params=pltpu.CompilerParams(needs_layout_passes=False,
                                             use_tc_tiling_on_sc=False),
    )
    return fn(colors_pad, edge_flat, batch, emb)


def _mlp_body(gx_ref, w1_ref, b1_ref, w2_ref, b2_ref, o_ref):
    h = jnp.dot(gx_ref[...], w1_ref[...],
                preferred_element_type=jnp.float32) + b1_ref[...]
    h = jnp.where(h > 0, h, jnp.float32(0.01) * h)
    logits = jnp.dot(h, w2_ref[...],
                     preferred_element_type=jnp.float32) + b2_ref[...]
    m = jnp.max(logits, axis=1, keepdims=True)
    s = logits - m
    lse = jnp.log(jnp.sum(jnp.exp(s), axis=1, keepdims=True))
    o_ref[...] = s - lse


def _mlp_tc(gx, W1, b1, W2, b2):
    return pl.pallas_call(
        _mlp_body,
        out_shape=jax.ShapeDtypeStruct((B, NCLASS), jnp.float32),
    )(gx, W1, b1, W2, b2)


@jax.jit
def kernel(x, edge_index, batch, emb, W1, b1, W2, b2):
    colors_pad = _colors_tc(x)
    gx = _sc_call(colors_pad, edge_index.reshape(-1), batch, emb)
    return _mlp_tc(gx, W1, b1, W2, b2)


# speedup vs baseline: 160.2332x; 1.3630x over previous
import functools

import jax
import jax.numpy as jnp
import numpy as np
from jax import lax
from jax.experimental import pallas as pl
from jax.experimental.pallas import tpu as pltpu
from jax.experimental.pallas import tpu_sc as plsc

N = 10000
E = 320000
NFEAT = 128
NHID = 64
NCLASS = 10
B = 256
EMB_ROWS = 5000
EMB_DIM = 32
NLAYER = 2

NT = 16
NPAD = 10240
NODES_PT = NPAD // NT
EPT = E // NT
BINS = 512
GPT = B // NT
VL = 16

_M1 = np.int32(np.uint32(0x85EBCA6B))
_M2 = np.int32(np.uint32(0xC2B2AE35))
_GOLD = np.int32(np.uint32(0x9E3779B1))


def _mix32(h):
    h = h * _M1
    h = h ^ lax.shift_right_logical(h, 13)
    h = h * _M2
    h = h ^ lax.shift_right_logical(h, 16)
    return h


def _ploop(n_total, unroll, body):
    plsc.parallel_loop(0, n_total, unroll=unroll)(
        lambda i: body(pl.multiple_of(i * VL, VL)))


def _argmax_body(x_ref, o_ref):
    am = jnp.argmax(x_ref[...], axis=-1).astype(jnp.int32)
    o_ref[...] = jnp.concatenate([am, jnp.zeros((NPAD - N,), jnp.int32)])


def _colors_tc(x):
    return pl.pallas_call(
        _argmax_body,
        out_shape=jax.ShapeDtypeStruct((NPAD,), jnp.int32),
    )(x)


def _sc_body(colors_hbm, edge_hbm, batch_hbm, emb_hbm, out_hbm,
             colors_v, mixed_v, agg_v, red_v, src_v, dst_v,
             batch_v, bins_v, binred_v, gidx_v, rows_v,
             spm_agg, spm_mixed, spm_bins, sem_s, sem_d, sem_b):
    wid = lax.axis_index("s")
    nbase = pl.multiple_of(wid * NODES_PT, 8)
    sbase = pl.multiple_of(wid * EPT, 8)
    dbase = pl.multiple_of(E + wid * EPT, 8)
    gbase = pl.multiple_of(wid * GPT, 8)

    cp_s = pltpu.async_copy(edge_hbm.at[pl.ds(sbase, EPT)], src_v, sem_s)
    cp_d = pltpu.async_copy(edge_hbm.at[pl.ds(dbase, EPT)], dst_v, sem_d)

    last = NT - 1

    @pl.when(wid < last)
    def _():
        pltpu.sync_copy(batch_hbm.at[pl.ds(nbase, NODES_PT)], batch_v)

    @pl.when(wid == last)
    def _():
        tail = N - last * NODES_PT
        pltpu.sync_copy(batch_hbm.at[pl.ds(last * NODES_PT, tail)],
                        batch_v.at[pl.ds(0, tail)])
        for u in range(tail // VL, NODES_PT // VL):
            batch_v[pl.ds(u * VL, VL)] = jnp.full((VL,), B, jnp.int32)

    pltpu.sync_copy(colors_hbm.at[pl.ds(nbase, NODES_PT)], colors_v)

    edges_pending = True
    for _layer in range(NLAYER):
        with jax.named_scope("mixshare"):
            def mix_body(o):
                mixed_v[pl.ds(nbase + o, VL)] = _mix32(colors_v[pl.ds(o, VL)])
            _ploop(NODES_PT // VL, 8, mix_body)

            pltpu.sync_copy(mixed_v.at[pl.ds(nbase, NODES_PT)],
                            spm_mixed.at[pl.ds(nbase, NODES_PT)])
            plsc.subcore_barrier()
            pltpu.sync_copy(spm_mixed, mixed_v)

        with jax.named_scope("zero"):
            def zero_body(o):
                agg_v[pl.ds(o, VL)] = jnp.zeros((VL,), jnp.int32)
            _ploop(NPAD // VL, 8, zero_body)

        if edges_pending:
            cp_s.wait()
            cp_d.wait()
            edges_pending = False

        with jax.named_scope("edges"):
            def edge_body(o):
                s = src_v[pl.ds(o, VL)]
                d = dst_v[pl.ds(o, VL)]
                vals = plsc.load_gather(mixed_v, [s])
                plsc.addupdate_scatter(agg_v, [d], vals)
            _ploop(EPT // VL, 8, edge_body)

        with jax.named_scope("reduce"):
            pltpu.sync_copy(agg_v, spm_agg.at[wid])
            plsc.subcore_barrier()
            cps = [pltpu.async_copy(spm_agg.at[t, pl.ds(nbase, NODES_PT)],
                                    red_v.at[t], sem_b)
                   for t in range(NT)]
            for cp in cps:
                cp.wait()

            def upd_body(o):
                a = red_v[0, pl.ds(o, VL)]
                for t in range(1, NT):
                    a = a + red_v[t, pl.ds(o, VL)]
                c = colors_v[pl.ds(o, VL)]
                colors_v[pl.ds(o, VL)] = _mix32((c * _GOLD) ^ a)
            _ploop(NODES_PT // VL, 4, upd_body)

    with jax.named_scope("bins"):
        def binzero_body(o):
            bins_v[pl.ds(o, VL)] = jnp.zeros((VL,), jnp.int32)
        _ploop(BINS // VL, 8, binzero_body)

        def bin_body(o):
            c = colors_v[pl.ds(o, VL)]
            b = batch_v[pl.ds(o, VL)]
            plsc.addupdate_scatter(bins_v, [b], _mix32(c))
        _ploop(NODES_PT // VL, 8, bin_body)

        pltpu.sync_copy(bins_v, spm_bins.at[wid])
        plsc.subcore_barrier()
        cps = [pltpu.async_copy(spm_bins.at[t, pl.ds(gbase, GPT)],
                                binred_v.at[t], sem_b)
               for t in range(NT)]
        for cp in cps:
            cp.wait()

        ghash = binred_v[0, pl.ds(0, GPT)]
        for t in range(1, NT):
            ghash = ghash + binred_v[t, pl.ds(0, GPT)]

        gu = plsc.bitcast(ghash, jnp.uint32)
        gidx_v[...] = (gu % jnp.uint32(EMB_ROWS)).astype(jnp.int32)

    with jax.named_scope("emb"):
        pltpu.async_copy(emb_hbm.at[gidx_v], rows_v, sem_s).wait()
        pltpu.sync_copy(rows_v, out_hbm.at[pl.ds(gbase, GPT)])


def _sc_call(colors_pad, edge_flat, batch, emb):
    mesh = plsc.VectorSubcoreMesh(core_axis_name="c", subcore_axis_name="s",
                                  num_cores=1)
    fn = pl.kernel(
        _sc_body,
        out_type=jax.ShapeDtypeStruct((B, EMB_DIM), jnp.float32),
        mesh=mesh,
        scratch_types=[
            pltpu.VMEM((NODES_PT,), jnp.int32),
            pltpu.VMEM((NPAD,), jnp.int32),
            pltpu.VMEM((NPAD,), jnp.int32),
            pltpu.VMEM((NT, NODES_PT), jnp.int32),
            pltpu.VMEM((EPT,), jnp.int32),
            pltpu.VMEM((EPT,), jnp.int32),
            pltpu.VMEM((NODES_PT,), jnp.int32),
            pltpu.VMEM((BINS,), jnp.int32),
            pltpu.VMEM((NT, GPT), jnp.int32),
            pltpu.VMEM((GPT,), jnp.int32),
            pltpu.VMEM((GPT, EMB_DIM), jnp.float32),
            pltpu.VMEM_SHARED((NT, NPAD), jnp.int32),
            pltpu.VMEM_SHARED((NPAD,), jnp.int32),
            pltpu.VMEM_SHARED((NT, BINS), jnp.int32),
            pltpu.SemaphoreType.DMA,
            pltpu.SemaphoreType.DMA,
            pltpu.SemaphoreType.DMA,
        ],
        compiler_params=pltpu.CompilerParams(needs_layout_passes=False,
                                             use_tc_tiling_on_sc=False),
    )
    return fn(colors_pad, edge_flat, batch, emb)


def _mlp_body(gx_ref, w1_ref, b1_ref, w2_ref, b2_ref, o_ref):
    h = jnp.dot(gx_ref[...], w1_ref[...],
                preferred_element_type=jnp.float32) + b1_ref[...]
    h = jnp.where(h > 0, h, jnp.float32(0.01) * h)
    logits = jnp.dot(h, w2_ref[...],
                     preferred_element_type=jnp.float32) + b2_ref[...]
    m = jnp.max(logits, axis=1, keepdims=True)
    s = logits - m
    lse = jnp.log(jnp.sum(jnp.exp(s), axis=1, keepdims=True))
    o_ref[...] = s - lse


def _mlp_tc(gx, W1, b1, W2, b2):
    return pl.pallas_call(
        _mlp_body,
        out_shape=jax.ShapeDtypeStruct((B, NCLASS), jnp.float32),
    )(gx, W1, b1, W2, b2)


@jax.jit
def kernel(x, edge_index, batch, emb, W1, b1, W2, b2):
    colors_pad = _colors_tc(x)
    gx = _sc_call(colors_pad, edge_index.reshape(-1), batch, emb)
    return _mlp_tc(gx, W1, b1, W2, b2)
